# Initial kernel scaffold; baseline (speedup 1.0000x reference)
import jax
import jax.numpy as jnp
from jax.experimental import pallas as pl


def kernel(idx, weight):
    raise NotImplementedError("write your pallas kernel here")


# speedup vs baseline: 1.1015x; 1.1015x over previous
import functools

import jax
import jax.numpy as jnp
from jax import lax
from jax.experimental import pallas as pl
from jax.experimental.pallas import tpu as pltpu
from jax.experimental.pallas import tpu_sc as plsc

EMBED_DIM = 32
NUM_CORES = 2
NUM_SUBCORES = 16
NUM_WORKERS = NUM_CORES * NUM_SUBCORES


@functools.partial(jax.jit, static_argnums=(2,))
def _embed_gather(idx_flat, weight, total):
    per_w = total // NUM_WORKERS
    chunk = 1600
    nchunk = per_w // chunk
    mesh = plsc.VectorSubcoreMesh(core_axis_name="c", subcore_axis_name="s")

    @functools.partial(
        pl.kernel,
        mesh=mesh,
        out_type=jax.ShapeDtypeStruct((total, EMBED_DIM), jnp.float32),
        scratch_types=[
            pltpu.VMEM((chunk,), jnp.int32),
            pltpu.VMEM((chunk, EMBED_DIM), jnp.float32),
            pltpu.SemaphoreType.DMA,
        ],
        compiler_params=pltpu.CompilerParams(use_tc_tiling_on_sc=False),
    )
    def run(idx_hbm, w_hbm, out_hbm, idx_v, rows_v, sem):
        wid = lax.axis_index("s") * NUM_CORES + lax.axis_index("c")
        base0 = wid * per_w

        def body(i, carry):
            base = base0 + i * chunk
            pltpu.sync_copy(idx_hbm.at[pl.ds(base, chunk)], idx_v)
            pltpu.async_copy(w_hbm.at[idx_v], rows_v, sem).wait()
            pltpu.sync_copy(rows_v, out_hbm.at[pl.ds(base, chunk)])
            return carry

        lax.fori_loop(0, nchunk, body, 0)

    return run(idx_flat, weight)


def kernel(idx, weight):
    b, s = idx.shape
    total = b * s
    idx_flat = idx.reshape(total).astype(jnp.int32)
    out = _embed_gather(idx_flat, weight, total)
    return out.reshape(b, s, EMBED_DIM)


# speedup vs baseline: 1.1092x; 1.0069x over previous
import functools

import jax
import jax.numpy as jnp
from jax import lax
from jax.experimental import pallas as pl
from jax.experimental.pallas import tpu as pltpu
from jax.experimental.pallas import tpu_sc as plsc

EMBED_DIM = 32
NUM_CORES = 2
NUM_SUBCORES = 16
NUM_WORKERS = NUM_CORES * NUM_SUBCORES
CHUNK = 800
NBUF = 4


@functools.partial(jax.jit, static_argnums=(2,))
def _embed_gather(idx_flat, weight, total):
    per_w = total // NUM_WORKERS
    nchunk = per_w // CHUNK
    nouter = nchunk // NBUF
    mesh = plsc.VectorSubcoreMesh(core_axis_name="c", subcore_axis_name="s")

    @functools.partial(
        pl.kernel,
        mesh=mesh,
        out_type=jax.ShapeDtypeStruct((total, EMBED_DIM), jnp.float32),
        scratch_types=[
            pltpu.VMEM((per_w,), jnp.int32),
            pltpu.VMEM((NBUF, CHUNK, EMBED_DIM), jnp.float32),
        ] + [pltpu.SemaphoreType.DMA] * (2 * NBUF),
        compiler_params=pltpu.CompilerParams(use_tc_tiling_on_sc=False),
    )
    def run(idx_hbm, w_hbm, out_hbm, idx_v, rows, *sems):
        gsems, ssems = sems[:NBUF], sems[NBUF:]
        wid = lax.axis_index("s") * NUM_CORES + lax.axis_index("c")
        base0 = wid * per_w

        def g_desc(g, b):
            return pltpu.make_async_copy(
                w_hbm.at[idx_v.at[pl.ds(g * CHUNK, CHUNK)]], rows.at[b], gsems[b])

        def s_desc(g, b):
            return pltpu.make_async_copy(
                rows.at[b], out_hbm.at[pl.ds(base0 + g * CHUNK, CHUNK)], ssems[b])

        pltpu.sync_copy(idx_hbm.at[pl.ds(base0, per_w)], idx_v)
        for b in range(NBUF):
            g_desc(b, b).start()

        def outer(t, carry):
            for b in range(NBUF):
                g = t * NBUF + b
                g_desc(g, b).wait()
                s_desc(g, b).start()
            for b in range(NBUF):
                g = t * NBUF + b

                @pl.when(t < nouter - 1)
                def _fire_next(g=g, b=b):
                    s_desc(g, b).wait()
                    g_desc(g + NBUF, b).start()

            return carry

        lax.fori_loop(0, nouter, outer, 0)
        for b in range(NBUF):
            s_desc((nouter - 1) * NBUF + b, b).wait()

    return run(idx_flat, weight)


def kernel(idx, weight):
    b, s = idx.shape
    total = b * s
    idx_flat = idx.reshape(total).astype(jnp.int32)
    out = _embed_gather(idx_flat, weight, total)
    return out.reshape(b, s, EMBED_DIM)


# speedup vs baseline: 1.5069x; 1.3585x over previous
import functools

import jax
import jax.numpy as jnp
from jax import lax
from jax.experimental import pallas as pl
from jax.experimental.pallas import tpu as pltpu
from jax.experimental.pallas import tpu_sc as plsc

EMBED_DIM = 32
SEQ = 50
BATCH = 16384
NUM_CORES = 2
NUM_SUBCORES = 16
NUM_WORKERS = NUM_CORES * NUM_SUBCORES
BLK = 128
NUM_UNITS = SEQ * (BATCH // BLK)
UNITS_PER_W = NUM_UNITS // NUM_WORKERS
NBUF = 2


@jax.jit
def _embed_gather(idx_t_flat, weight):
    mesh = plsc.VectorSubcoreMesh(core_axis_name="c", subcore_axis_name="s")

    @functools.partial(
        pl.kernel,
        mesh=mesh,
        out_type=jax.ShapeDtypeStruct((SEQ, 4, BATCH // BLK, 8, BLK), jnp.float32),
        scratch_types=[
            pltpu.VMEM((UNITS_PER_W * BLK,), jnp.int32),
            pltpu.VMEM((NBUF, BLK, EMBED_DIM), jnp.float32),
            pltpu.VMEM((NBUF, 4, 8, BLK), jnp.float32),
        ] + [pltpu.SemaphoreType.DMA] * (2 * NBUF),
        compiler_params=pltpu.CompilerParams(
            use_tc_tiling_on_sc=False, needs_layout_passes=False),
    )
    def run(idx_hbm, w_hbm, out_hbm, idx_v, rows, tiles, *sems):
        gsems, ssems = sems[:NBUF], sems[NBUF:]
        wid = lax.axis_index("s") * NUM_CORES + lax.axis_index("c")
        u0 = wid * UNITS_PER_W
        iota = lax.iota(jnp.int32, 16)

        def g_desc(uu, b):
            return pltpu.make_async_copy(
                w_hbm.at[idx_v.at[pl.ds(uu * BLK, BLK)]], rows.at[b], gsems[b])

        def s_descs(uu, b):
            u = u0 + uu
            s, tc = u // (BATCH // BLK), u % (BATCH // BLK)
            return [
                pltpu.make_async_copy(
                    tiles.at[b, tr], out_hbm.at[s, tr, tc], ssems[b])
                for tr in range(4)
            ]

        pltpu.sync_copy(idx_hbm.at[pl.ds(u0 * BLK, UNITS_PER_W * BLK)], idx_v)
        for b in range(NBUF):
            g_desc(b, b).start()

        def body(t, carry):
            for b in range(NBUF):
                uu = t * NBUF + b

                @pl.when(t > 0)
                def _drain_store(uu=uu, b=b):
                    for d in s_descs(uu, b):
                        d.wait()

                g_desc(uu, b).wait()
                for c in range(EMBED_DIM):
                    cvec = jnp.full((16,), c, jnp.int32)
                    for j in range(BLK // 16):
                        v = plsc.load_gather(
                            rows.at[b], [iota + (j * 16), cvec])
                        tiles.at[b][c // 8, c % 8, pl.ds(j * 16, 16)] = v
                for d in s_descs(uu, b):
                    d.start()

                @pl.when(uu + NBUF < UNITS_PER_W)
                def _fire_next(uu=uu, b=b):
                    g_desc(uu + NBUF, b).start()

            return carry

        lax.fori_loop(0, UNITS_PER_W // NBUF, body, 0)
        for b in range(NBUF):
            for d in s_descs(UNITS_PER_W - NBUF + b, b):
                d.wait()

    return run(idx_t_flat, weight)


def kernel(idx, weight):
    idx_t_flat = idx.T.reshape(-1).astype(jnp.int32)
    out5 = _embed_gather(idx_t_flat, weight)
    return out5.transpose(2, 4, 0, 1, 3).reshape(BATCH, SEQ, EMBED_DIM)
